# Initial kernel scaffold; baseline (speedup 1.0000x reference)
import jax
import jax.numpy as jnp
from jax.experimental import pallas as pl


def kernel(x, edge_index, W1, b1, W2, b2):
    raise NotImplementedError("write your pallas kernel here")


# speedup vs baseline: 30.7939x; 30.7939x over previous
import functools

import jax
import jax.numpy as jnp
from jax import lax
from jax.experimental import pallas as pl
from jax.experimental.pallas import tpu as pltpu
from jax.experimental.pallas import tpu_sc as plsc

N = 10000
E = 320000
D = 128
NC = 2
NS = 16
NW = NC * NS
C = 80
CS = 80
EPW = E // NW
CHUNKS = EPW // CS
EROWS = E // C
NP = 10240
DEG_ZPT = NP // NS
DEG_RPW = NP // NW
ZB = 32
ACC_ZPT = NP // NS

_MAGIC = 0x5F3759DF


def _zero_f32(ref, num):
    def body(k, _):
        ref[pl.ds(k * 16, 16)] = jnp.zeros((16,), jnp.float32)
        return 0
    lax.fori_loop(0, num, body, 0)


def _mesh():
    return plsc.VectorSubcoreMesh(core_axis_name="c", subcore_axis_name="s")


@functools.partial(
    pl.kernel,
    mesh=_mesh(),
    compiler_params=pltpu.CompilerParams(use_tc_tiling_on_sc=False),
    out_type=jax.ShapeDtypeStruct((NP,), jnp.float32),
    scratch_types=[
        pltpu.VMEM((EROWS // NS, C), jnp.int32),
        pltpu.VMEM((C,), jnp.float32),
        pltpu.VMEM((DEG_ZPT,), jnp.float32),
        pltpu.VMEM((DEG_RPW,), jnp.float32),
        pltpu.VMEM((DEG_RPW,), jnp.float32),
        pltpu.VMEM_SHARED((NP,), jnp.float32),
    ],
)
def _deg_kernel(dst_hbm, dinv_hbm, idx_v, ones_v, zb_v, degv, dinvv, deg_sp):
    c = lax.axis_index("c")
    s = lax.axis_index("s")
    wid = c * NS + s

    _zero_f32(zb_v, DEG_ZPT // 16)

    def ones_body(k, _):
        ones_v[pl.ds(k * 16, 16)] = jnp.ones((16,), jnp.float32)
        return 0
    lax.fori_loop(0, C // 16, ones_body, 0)

    pltpu.sync_copy(zb_v, deg_sp.at[pl.ds(s * DEG_ZPT, DEG_ZPT)])

    rows = EROWS // NS
    pltpu.sync_copy(dst_hbm.at[s], idx_v)

    plsc.subcore_barrier()

    def scat_body(j, _):
        pltpu.sync_copy(ones_v, deg_sp.at[idx_v.at[j]], add=True)
        return 0
    lax.fori_loop(0, rows, scat_body, 0)

    plsc.subcore_barrier()

    base = wid * DEG_RPW
    pltpu.sync_copy(deg_sp.at[pl.ds(base, DEG_RPW)], degv)

    def newton_body(k, _):
        d = degv[pl.ds(k * 16, 16)] + 1.0
        i = jnp.int32(_MAGIC) - lax.shift_right_arithmetic(
            lax.bitcast_convert_type(d, jnp.int32), 1)
        y = lax.bitcast_convert_type(i, jnp.float32)
        y = y * (1.5 - 0.5 * d * y * y)
        y = y * (1.5 - 0.5 * d * y * y)
        y = y * (1.5 - 0.5 * d * y * y)
        dinvv[pl.ds(k * 16, 16)] = y
        return 0
    lax.fori_loop(0, DEG_RPW // 16, newton_body, 0)

    pltpu.sync_copy(dinvv, dinv_hbm.at[pl.ds(base, DEG_RPW)])


@functools.partial(
    pl.kernel,
    mesh=_mesh(),
    compiler_params=pltpu.CompilerParams(use_tc_tiling_on_sc=False),
    out_type=jax.ShapeDtypeStruct((NC, NP, D), jnp.float32),
    scratch_types=[
        pltpu.VMEM((CHUNKS, CS), jnp.int32),
        pltpu.VMEM((CHUNKS, CS), jnp.int32),
        pltpu.VMEM((CS, D), jnp.float32),
        pltpu.VMEM((CS, D), jnp.float32),
        pltpu.VMEM((ZB, D), jnp.float32),
        pltpu.VMEM_SHARED((NP, D), jnp.float32),
        pltpu.SemaphoreType.DMA,
        pltpu.SemaphoreType.DMA,
    ],
)
def _scatter_kernel(y_hbm, src_hbm, dst_hbm, out_hbm,
                    sidx, didx, rows0, rows1, zb, acc_sp, sem0, sem1):
    c = lax.axis_index("c")
    s = lax.axis_index("s")
    wid = c * NS + s

    def zb_body(r, _):
        for cc in range(D // 16):
            zb[r, pl.ds(cc * 16, 16)] = jnp.zeros((16,), jnp.float32)
        return 0
    lax.fori_loop(0, ZB, zb_body, 0)

    def zinit_body(k, _):
        pltpu.sync_copy(zb, acc_sp.at[pl.ds(s * ACC_ZPT + k * ZB, ZB)])
        return 0
    lax.fori_loop(0, ACC_ZPT // ZB, zinit_body, 0)

    pltpu.sync_copy(src_hbm.at[wid], sidx)
    pltpu.sync_copy(dst_hbm.at[wid], didx)

    pltpu.async_copy(y_hbm.at[sidx.at[0]], rows0, sem0)
    pltpu.async_copy(y_hbm.at[sidx.at[1]], rows1, sem1)

    plsc.subcore_barrier()

    def step(j, buf, sem):
        pltpu.make_async_copy(y_hbm.at[sidx.at[j]], buf, sem).wait()
        pltpu.sync_copy(buf, acc_sp.at[didx.at[j]], add=True)
        nxt = j + 2
        @pl.when(nxt < CHUNKS)
        def _():
            pltpu.async_copy(y_hbm.at[sidx.at[nxt]], buf, sem)

    def loop_body(g, _):
        step(2 * g, rows0, sem0)
        step(2 * g + 1, rows1, sem1)
        return 0
    lax.fori_loop(0, CHUNKS // 2, loop_body, 0)
    if CHUNKS % 2:
        step(CHUNKS - 1, rows0, sem0)

    plsc.subcore_barrier()

    pltpu.sync_copy(acc_sp.at[pl.ds(s * ACC_ZPT, ACC_ZPT)],
                    out_hbm.at[c, pl.ds(s * ACC_ZPT, ACC_ZPT)])


def _tc1_body(x_ref, w_ref, dinv_ref, y_ref):
    y_ref[...] = jnp.dot(x_ref[...], w_ref[...],
                         preferred_element_type=jnp.float32) * dinv_ref[...]


def _tc2_body(acc_ref, y1_ref, dinv_ref, b1_ref, w2_ref, y2_ref):
    agg = acc_ref[0, pl.ds(0, N)] + acc_ref[1, pl.ds(0, N)]
    pre = (agg + y1_ref[...]) * dinv_ref[...] + b1_ref[...]
    h = jnp.maximum(pre, 0.0)
    y2_ref[...] = jnp.dot(h, w2_ref[...],
                          preferred_element_type=jnp.float32) * dinv_ref[...]


def _tc3_body(acc_ref, y2_ref, dinv_ref, b2_ref, o_ref):
    agg = acc_ref[0, pl.ds(0, N)] + acc_ref[1, pl.ds(0, N)]
    o_ref[...] = (agg + y2_ref[...]) * dinv_ref[...] + b2_ref[...]


_f32 = jnp.float32


def kernel(x, edge_index, W1, b1, W2, b2):
    src = edge_index[0].astype(jnp.int32).reshape(NW, CHUNKS, CS)
    dst = edge_index[1].astype(jnp.int32).reshape(NW, CHUNKS, CS)
    dst_per_tile = dst.reshape(NS, EROWS // NS, C)

    dinv_pad = _deg_kernel(dst_per_tile)
    dinv2 = dinv_pad[:N].reshape(N, 1)

    y1 = pl.pallas_call(
        _tc1_body,
        out_shape=jax.ShapeDtypeStruct((N, D), _f32),
    )(x, W1, dinv2)

    acc1 = _scatter_kernel(y1, src, dst)

    y2 = pl.pallas_call(
        _tc2_body,
        out_shape=jax.ShapeDtypeStruct((N, D), _f32),
    )(acc1, y1, dinv2, b1.reshape(1, D), W2)

    acc2 = _scatter_kernel(y2, src, dst)

    out = pl.pallas_call(
        _tc3_body,
        out_shape=jax.ShapeDtypeStruct((N, D), _f32),
    )(acc2, y2, dinv2, b2.reshape(1, D))
    return out
